# baseline (speedup 1.0000x reference)
import functools

import jax
import jax.numpy as jnp
from jax import lax
from jax.experimental import pallas as pl
from jax.experimental.pallas import tpu as pltpu
from jax.experimental.pallas import tpu_sc as plsc

NC = 2
NS = 16
NW = NC * NS
K_CH = 128

F32 = jnp.float32


def _dot(a, w):
    return lax.dot_general(
        a, w, (((1,), (1,)), ((), ())),
        preferred_element_type=F32, precision=lax.Precision.HIGHEST)


def _elu(x):
    return jnp.where(x > 0, x, jnp.exp(x) - 1.0)


def _sigmoid(x):
    return 1.0 / (1.0 + jnp.exp(-x))


def _relu(x):
    return jnp.maximum(x, 0.0)


def _matmul_body(x_ref, w_ref, o_ref):
    o_ref[...] = _dot(x_ref[...], w_ref[...])


def _matmul(x, w, bm=256):
    m, k = x.shape
    n = w.shape[0]
    return pl.pallas_call(
        _matmul_body,
        grid=(m // bm,),
        in_specs=[pl.BlockSpec((bm, k), lambda i: (i, 0)),
                  pl.BlockSpec((n, k), lambda i: (0, 0))],
        out_specs=pl.BlockSpec((bm, n), lambda i: (i, 0)),
        out_shape=jax.ShapeDtypeStruct((m, n), F32),
    )(x, w)


def _combine_next_body(xs_ref, agg0_ref, agg1_ref, deg0_ref, deg1_ref,
                       b_ref, ws_ref, wn_ref, ys_ref, yw_ref):
    n, w_in = xs_ref.shape
    agg = agg0_ref[:, :w_in] + agg1_ref[:, :w_in]
    deg = jnp.maximum(deg0_ref[:, 0:1] + deg1_ref[:, 0:1], 1.0)
    h = _elu(xs_ref[...] + agg / deg + b_ref[...])
    ys_ref[...] = _dot(h, ws_ref[...])
    yw_ref[...] = _dot(h, wn_ref[...])


def _fused_specs(n, ntot, off):
    i0 = off // n
    i1 = (ntot + off) // n
    assert i0 * n == off and i1 * n == ntot + off
    return [pl.BlockSpec((n, 128), lambda i, k=i0: (k, 0)),
            pl.BlockSpec((n, 128), lambda i, k=i1: (k, 0))]


def _full(shape):
    return pl.BlockSpec(shape, lambda i: (0,) * len(shape))


def _combine_next(xs, agg, degp, off, b, ws, wn_pad):
    n, w_in = xs.shape
    w_out = ws.shape[0]
    ntot = agg.shape[0] // 2
    ab = _fused_specs(n, ntot, off)
    out = pl.pallas_call(
        _combine_next_body,
        grid=(1,),
        in_specs=[_full((n, w_in))] + ab + ab
        + [_full((1, w_in)), _full((w_out, w_in)), _full((128, w_in))],
        out_specs=[_full((n, w_out)), _full((n, 128))],
        out_shape=[jax.ShapeDtypeStruct((n, w_out), F32),
                   jax.ShapeDtypeStruct((n, 128), F32)],
    )(xs, agg, agg, degp, degp, b.reshape(1, -1), ws, wn_pad)
    return out


def _combine_nlf_body(xs_ref, agg0_ref, agg1_ref, deg0_ref, deg1_ref,
                      b_ref, resp_ref, resb_ref,
                      f1a_ref, f1b_ref, f1bb_ref, f2_ref, f2b_ref, o_ref):
    n, w_in = xs_ref.shape
    agg = agg0_ref[:, :w_in] + agg1_ref[:, :w_in]
    deg = jnp.maximum(deg0_ref[:, 0:1] + deg1_ref[:, 0:1], 1.0)
    h = _elu(xs_ref[...] + agg / deg + b_ref[...])
    res = _elu(resp_ref[...] + resb_ref[...])
    t = _relu(_dot(h, f1a_ref[...]) + _dot(res, f1b_ref[...]) + f1bb_ref[...])
    att = _sigmoid(_dot(t, f2_ref[...]) + f2b_ref[...])
    o_ref[...] = att * h + (1.0 - att) * res


def _rep_gate(w, b, lanes=64):
    return jnp.tile(w, (lanes, 1)), jnp.tile(b.reshape(1, 1), (1, lanes))


def _combine_nlf(xs, agg, degp, off, b, res_pre, res_b, p, name):
    n, w = xs.shape
    f2, f2b = _rep_gate(p[name + '_fc2_w'], p[name + '_fc2_b'], w)
    ntot = agg.shape[0] // 2
    ab = _fused_specs(n, ntot, off)
    return pl.pallas_call(
        _combine_nlf_body,
        grid=(1,),
        in_specs=[_full((n, w))] + ab + ab
        + [_full((1, w)), _full((n, w)), _full((1, w)),
           _full((w, w)), _full((w, w)), _full((1, w)),
           _full((w, w)), _full((1, w))],
        out_specs=_full((n, w)),
        out_shape=jax.ShapeDtypeStruct((n, w), F32),
    )(xs, agg, agg, degp, degp, b.reshape(1, -1), res_pre,
      res_b.reshape(1, -1),
      p[name + '_fc1_w'][:, :w], p[name + '_fc1_w'][:, w:],
      p[name + '_fc1_b'].reshape(1, -1),
      f2, f2b)


def _nlf_gf_body(x1_ref, x2_ref, f1a_ref, f1b_ref, f1bb_ref, f2_ref, f2b_ref,
                 g1_ref, g1b_ref, g2_ref, g2b_ref, o_ref):
    x1 = x1_ref[...]
    x2 = x2_ref[...]
    t = _relu(_dot(x1, f1a_ref[...]) + _dot(x2, f1b_ref[...]) + f1bb_ref[...])
    att = _sigmoid(_dot(t, f2_ref[...]) + f2b_ref[...])
    y = att * x1 + (1.0 - att) * x2
    gc = jnp.mean(y, axis=0, keepdims=True)
    hg = _relu(_dot(gc, g1_ref[...]) + g1b_ref[...])
    ag = _sigmoid(_dot(hg, g2_ref[...]) + g2b_ref[...])
    o_ref[...] = ag * y + (1.0 - ag) * gc


def _nlf_gf(x1, x2, p, nlf_name, gf_name):
    n, w = x1.shape
    f2, f2b = _rep_gate(p[nlf_name + '_fc2_w'], p[nlf_name + '_fc2_b'], w)
    g2, g2b = _rep_gate(p[gf_name + '_fc2_w'], p[gf_name + '_fc2_b'], w)
    return pl.pallas_call(
        _nlf_gf_body,
        out_shape=jax.ShapeDtypeStruct((n, w), F32),
    )(x1, x2,
      p[nlf_name + '_fc1_w'][:, :w], p[nlf_name + '_fc1_w'][:, w:],
      p[nlf_name + '_fc1_b'].reshape(1, -1),
      f2, f2b,
      p[gf_name + '_fc1_w'], p[gf_name + '_fc1_b'].reshape(1, -1),
      g2, g2b)


def _mlp_body(x_ref, w0_ref, b0_ref, w1_ref, b1_ref, o_ref):
    h = _elu(_dot(x_ref[...], w0_ref[...]) + b0_ref[...])
    o_ref[...] = _sigmoid(_dot(h, w1_ref[...]) + b1_ref[...])


def _mlp(x, w0, b0, w1, b1, bm=2048):
    m, k = x.shape
    h0 = w0.shape[0]
    return pl.pallas_call(
        _mlp_body,
        grid=(m // bm,),
        in_specs=[pl.BlockSpec((bm, k), lambda i: (i, 0)),
                  pl.BlockSpec((h0, k), lambda i: (0, 0)),
                  pl.BlockSpec((1, h0), lambda i: (0, 0)),
                  pl.BlockSpec((128, h0), lambda i: (0, 0)),
                  pl.BlockSpec((1, 128), lambda i: (0, 0))],
        out_specs=pl.BlockSpec((bm, 128), lambda i: (i, 0)),
        out_shape=jax.ShapeDtypeStruct((m, 128), F32),
    )(x, w0, b0.reshape(1, -1), *_rep_gate(w1, b1, 128))[:, :1]


_SC_MESH = plsc.VectorSubcoreMesh(core_axis_name="c", subcore_axis_name="s",
                                  num_cores=NC, num_subcores=NS)


@functools.lru_cache(maxsize=None)
def _segsum_call(n_nodes, width, n_edges):
    NSLOT = 3
    ew = n_edges // NW
    nch = ew // K_CH
    rps = n_nodes // NS
    assert ew * NW == n_edges and nch * K_CH == ew and rps * NS == n_nodes

    scratch = (
        [pltpu.VMEM((K_CH,), jnp.int32) for _ in range(nch)]
        + [pltpu.VMEM((K_CH,), jnp.int32) for _ in range(nch)]
        + [pltpu.VMEM((NSLOT, K_CH, width), F32),
           pltpu.VMEM_SHARED((n_nodes, width), F32)]
        + [pltpu.SemaphoreType.DMA]
        + [pltpu.SemaphoreType.DMA for _ in range(NSLOT)]
        + [pltpu.SemaphoreType.DMA for _ in range(NSLOT)])

    def body(xw_hbm, src_hbm, dst_hbm, z_hbm, agg_hbm, *scr):
        sidx = scr[:nch]
        didx = scr[nch:2 * nch]
        rows, sh_agg, isem = scr[2 * nch:2 * nch + 3]
        gsem = scr[2 * nch + 3:2 * nch + 3 + NSLOT]
        ssem = scr[2 * nch + 3 + NSLOT:2 * nch + 3 + 2 * NSLOT]
        c = lax.axis_index("c")
        s = lax.axis_index("s")
        wid = c * NS + s
        r0 = s * rps
        pltpu.sync_copy(z_hbm, sh_agg.at[pl.ds(r0, rps)])
        plsc.subcore_barrier()

        base = wid * ew

        ih = [pltpu.async_copy(src_hbm.at[pl.ds(base + i * K_CH, K_CH)],
                               sidx[i], isem) for i in range(nch)]
        ih += [pltpu.async_copy(dst_hbm.at[pl.ds(base + i * K_CH, K_CH)],
                                didx[i], isem) for i in range(nch)]
        for h in ih:
            h.wait()

        gh = [None] * NSLOT
        for b in range(min(NSLOT, nch)):
            gh[b] = pltpu.async_copy(xw_hbm.at[sidx[b]], rows.at[b], gsem[b])
        tail_s = [None] * NSLOT
        for i in range(nch):
            b = i % NSLOT
            gh[b].wait()
            sh = pltpu.async_copy(rows.at[b], sh_agg.at[didx[i]],
                                  ssem[b], add=True)
            j = i + NSLOT
            if j < nch:
                sh.wait()
                gh[b] = pltpu.async_copy(xw_hbm.at[sidx[j]], rows.at[b],
                                         gsem[b])
            else:
                tail_s[b] = sh
        for h in tail_s:
            if h is not None:
                h.wait()
        plsc.subcore_barrier()
        pltpu.sync_copy(sh_agg.at[pl.ds(r0, rps)],
                        agg_hbm.at[pl.ds(c * n_nodes + r0, rps)])

    return pl.kernel(
        body, out_type=jax.ShapeDtypeStruct((2 * n_nodes, width), F32),
        mesh=_SC_MESH, scratch_types=scratch)


def _segsum(xw, src, dst, n_nodes):
    n_edges = src.shape[0]
    width = xw.shape[1]
    z = jnp.zeros((n_nodes // NS, width), F32)
    return _segsum_call(n_nodes, width, n_edges)(xw, src, dst, z)


@functools.lru_cache(maxsize=None)
def _deg_call(n_nodes, n_edges):
    NSLOT = 4
    ew = n_edges // NW
    nch = ew // K_CH
    rps = n_nodes // NS
    assert ew * NW == n_edges and nch * K_CH == ew and rps * NS == n_nodes

    def body(dst_hbm, z_hbm, ones_hbm, deg_hbm, *scr):
        didx = scr[:nch]
        ones_v, sh_deg, isem = scr[nch:nch + 3]
        dsem = scr[nch + 3:]
        c = lax.axis_index("c")
        s = lax.axis_index("s")
        wid = c * NS + s
        r0 = s * rps
        pltpu.sync_copy(z_hbm, sh_deg.at[pl.ds(r0, rps)])
        pltpu.sync_copy(ones_hbm, ones_v)
        plsc.subcore_barrier()

        base = wid * ew
        ih = [pltpu.async_copy(dst_hbm.at[pl.ds(base + i * K_CH, K_CH)],
                               didx[i], isem) for i in range(nch)]
        for h in ih:
            h.wait()
        dh = [None] * NSLOT
        for i in range(nch):
            b = i % NSLOT
            if dh[b] is not None:
                dh[b].wait()
            dh[b] = pltpu.async_copy(ones_v, sh_deg.at[didx[i]],
                                     dsem[b], add=True)
        for h in dh:
            if h is not None:
                h.wait()
        plsc.subcore_barrier()
        pltpu.sync_copy(sh_deg.at[pl.ds(r0, rps)],
                        deg_hbm.at[pl.ds(c * n_nodes + r0, rps)])

    return pl.kernel(
        body, out_type=jax.ShapeDtypeStruct((2 * n_nodes, 128), F32),
        mesh=_SC_MESH,
        scratch_types=(
            [pltpu.VMEM((K_CH,), jnp.int32) for _ in range(nch)]
            + [pltpu.VMEM((K_CH, 128), F32),
               pltpu.VMEM_SHARED((n_nodes, 128), F32),
               pltpu.SemaphoreType.DMA]
            + [pltpu.SemaphoreType.DMA for _ in range(NSLOT)]))


def _deg(dst, n_nodes):
    z = jnp.zeros((n_nodes // NS, 128), F32)
    ones = jnp.ones((K_CH, 128), F32)
    return _deg_call(n_nodes, dst.shape[0])(dst, z, ones)


@functools.lru_cache(maxsize=None)
def _gather_call(n_rows, n_tab, width):
    NSLOT = 4
    per_w = n_rows // NW
    nch = per_w // K_CH
    assert per_w * NW == n_rows and nch * K_CH == per_w

    def body(tab_hbm, idx_hbm, out_hbm, *scr):
        idxs = scr[:nch]
        rows, isem = scr[nch:nch + 2]
        gsem = scr[nch + 2:nch + 2 + NSLOT]
        wsem = scr[nch + 2 + NSLOT:]
        wid = lax.axis_index("c") * NS + lax.axis_index("s")
        base = wid * per_w

        ih = [pltpu.async_copy(idx_hbm.at[pl.ds(base + i * K_CH, K_CH)],
                               idxs[i], isem) for i in range(nch)]
        for h in ih:
            h.wait()

        gh = [None] * NSLOT
        for b in range(min(NSLOT, nch)):
            gh[b] = pltpu.async_copy(tab_hbm.at[idxs[b]], rows.at[b], gsem[b])
        tail = [None] * NSLOT
        for i in range(nch):
            b = i % NSLOT
            gh[b].wait()
            wh = pltpu.async_copy(rows.at[b],
                                  out_hbm.at[pl.ds(base + i * K_CH, K_CH)],
                                  wsem[b])
            j = i + NSLOT
            if j < nch:
                wh.wait()
                gh[b] = pltpu.async_copy(tab_hbm.at[idxs[j]], rows.at[b],
                                         gsem[b])
            else:
                tail[b] = wh
        for h in tail:
            if h is not None:
                h.wait()

    return pl.kernel(
        body,
        out_type=jax.ShapeDtypeStruct((n_rows, width), F32),
        mesh=_SC_MESH,
        scratch_types=(
            [pltpu.VMEM((K_CH,), jnp.int32) for _ in range(nch)]
            + [pltpu.VMEM((NSLOT, K_CH, width), F32),
               pltpu.SemaphoreType.DMA]
            + [pltpu.SemaphoreType.DMA for _ in range(NSLOT)]
            + [pltpu.SemaphoreType.DMA for _ in range(NSLOT)]))


def _gather_rows(table, idx):
    return _gather_call(idx.shape[0], table.shape[0], table.shape[1])(
        table, idx)


def _pad_rows(w, to=128):
    n, k = w.shape
    if n == to:
        return w
    return jnp.concatenate([w, jnp.zeros((to - n, k), F32)], axis=0)


def kernel(miRNA, disease, img, params, mm_edge_index, dd_edge_index,
           md_edge_index, samples, triplet_samples):
    p = params
    nm = miRNA.shape[0]
    nd = disease.shape[0]

    wm = jnp.concatenate([p['lin_m_w'], p['res1_w'],
                          p['mm1_ws'], p['mm1_wn']], axis=0)
    wd = jnp.concatenate([p['lin_d_w'], p['res2_w'],
                          p['dd1_ws'], p['dd1_wn']], axis=0)
    am = _matmul(miRNA, wm)
    ad = _matmul(disease, wd)
    lm, resm_pre, xs1_m, xw1_m = (am[:, :512], am[:, 512:576],
                                  am[:, 576:704], am[:, 704:832])
    ld, resd_pre, xs1_d, xw1_d = (ad[:, :512], ad[:, 512:576],
                                  ad[:, 576:704], ad[:, 704:832])

    md_feat = jnp.concatenate([lm, ld], axis=0)
    wmd = jnp.concatenate([p['res3_w'], p['md1_ws'], p['md1_wn']], axis=0)
    amd = _matmul(md_feat, wmd)
    resmd_pre, xs1_g, xw1_g = amd[:, :64], amd[:, 64:192], amd[:, 192:320]

    ntot = 2 * (nm + nd)
    o_mm, o_dd, o_md = 0, nm, nm + nd
    csrc = jnp.concatenate([mm_edge_index[0], dd_edge_index[0] + o_dd,
                            md_edge_index[0] + o_md])
    cdst = jnp.concatenate([mm_edge_index[1], dd_edge_index[1] + o_dd,
                            md_edge_index[1] + o_md])
    degp = _deg(cdst, ntot)

    agg1 = _segsum(jnp.concatenate([xw1_m, xw1_d, xw1_g]), csrc, cdst, ntot)
    ys2_m, yw2_m = _combine_next(xs1_m, agg1, degp, o_mm, p['mm1_b'],
                                 p['mm2_ws'], _pad_rows(p['mm2_wn']))
    ys2_d, yw2_d = _combine_next(xs1_d, agg1, degp, o_dd, p['dd1_b'],
                                 p['dd2_ws'], _pad_rows(p['dd2_wn']))
    ys2_g, yw2_g = _combine_next(xs1_g, agg1, degp, o_md, p['md1_b'],
                                 p['md2_ws'], _pad_rows(p['md2_wn']))

    agg2 = _segsum(jnp.concatenate([yw2_m, yw2_d, yw2_g]), csrc, cdst, ntot)
    ys3_m, yw3_m = _combine_next(ys2_m, agg2, degp, o_mm, p['mm2_b'],
                                 p['mm3_ws'], _pad_rows(p['mm3_wn']))
    ys3_d, yw3_d = _combine_next(ys2_d, agg2, degp, o_dd, p['dd2_b'],
                                 p['dd3_ws'], _pad_rows(p['dd3_wn']))
    ys3_g, yw3_g = _combine_next(ys2_g, agg2, degp, o_md, p['md2_b'],
                                 p['md3_ws'], _pad_rows(p['md3_wn']))

    agg3 = _segsum(jnp.concatenate([yw3_m, yw3_d, yw3_g]), csrc, cdst, ntot)
    emb_mm_sim = _combine_nlf(ys3_m, agg3, degp, o_mm, p['mm3_b'],
                              resm_pre, p['res1_b'], p, 'nlf_m')
    emb_dd_sim = _combine_nlf(ys3_d, agg3, degp, o_dd, p['dd3_b'],
                              resd_pre, p['res2_b'], p, 'nlf_d')
    emb_ass = _combine_nlf(ys3_g, agg3, degp, o_md, p['md3_b'],
                           resmd_pre, p['res3_b'], p, 'nlf_md')

    emb_mm = _nlf_gf(emb_mm_sim, emb_ass[:nm], p, 'nlf_m', 'gf_m')
    emb_dd = _nlf_gf(emb_dd_sim, emb_ass[nm:], p, 'nlf_d', 'gf_d')

    emb_cat = jnp.concatenate([
        jnp.concatenate([emb_mm, emb_dd], axis=0),
        jnp.zeros((nm + nd, 64), F32)], axis=1)
    off = jnp.int32(nm)
    idx_cat = jnp.concatenate([
        samples[:, 0], samples[:, 1] + off,
        triplet_samples[:, 0], triplet_samples[:, 1] + off,
        triplet_samples[:, 2] + off])
    g = _gather_rows(emb_cat, idx_cat)[:, :64]
    ns = samples.shape[0]
    gm, gd = g[:ns], g[ns:2 * ns]
    anchor = g[2 * ns:3 * ns]
    pos = g[3 * ns:4 * ns]
    neg = g[4 * ns:5 * ns]

    emb = jnp.concatenate([gm, gd, img[:ns]], axis=1)
    out = _mlp(emb, p['mlp0_w'], p['mlp0_b'], p['mlp1_w'], p['mlp1_b'])
    return (out, anchor, pos, neg)


# speedup vs baseline: 1.0636x; 1.0636x over previous
import functools

import jax
import jax.numpy as jnp
from jax import lax
from jax.experimental import pallas as pl
from jax.experimental.pallas import tpu as pltpu
from jax.experimental.pallas import tpu_sc as plsc

NC = 2
NS = 16
NW = NC * NS
K_CH = 128

F32 = jnp.float32


def _dot(a, w):
    return lax.dot_general(
        a, w, (((1,), (1,)), ((), ())),
        preferred_element_type=F32, precision=lax.Precision.HIGHEST)


def _elu(x):
    return jnp.where(x > 0, x, jnp.exp(x) - 1.0)


def _sigmoid(x):
    return 1.0 / (1.0 + jnp.exp(-x))


def _relu(x):
    return jnp.maximum(x, 0.0)


def _matmul_body(x_ref, w_ref, o_ref):
    o_ref[...] = _dot(x_ref[...], w_ref[...])


def _matmul(x, w, bm=256):
    m, k = x.shape
    n = w.shape[0]
    return pl.pallas_call(
        _matmul_body,
        grid=(m // bm,),
        in_specs=[pl.BlockSpec((bm, k), lambda i: (i, 0)),
                  pl.BlockSpec((n, k), lambda i: (0, 0))],
        out_specs=pl.BlockSpec((bm, n), lambda i: (i, 0)),
        out_shape=jax.ShapeDtypeStruct((m, n), F32),
    )(x, w)


def _combine_next_body(xs_ref, agg0_ref, agg1_ref, deg0_ref, deg1_ref,
                       b_ref, ws_ref, wn_ref, ys_ref, yw_ref):
    n, w_in = xs_ref.shape
    agg = agg0_ref[:, :w_in] + agg1_ref[:, :w_in]
    deg = jnp.maximum(deg0_ref[:, 0:1] + deg1_ref[:, 0:1], 1.0)
    h = _elu(xs_ref[...] + agg / deg + b_ref[...])
    ys_ref[...] = _dot(h, ws_ref[...])
    yw_ref[...] = _dot(h, wn_ref[...])


def _fused_specs(n, ntot, off):
    i0 = off // n
    i1 = (ntot + off) // n
    assert i0 * n == off and i1 * n == ntot + off
    return [pl.BlockSpec((n, 128), lambda i, k=i0: (k, 0)),
            pl.BlockSpec((n, 128), lambda i, k=i1: (k, 0))]


def _full(shape):
    return pl.BlockSpec(shape, lambda i: (0,) * len(shape))


def _combine_next(xs, agg, degp, off, b, ws, wn_pad):
    n, w_in = xs.shape
    w_out = ws.shape[0]
    ab = _fused_specs(n, agg.shape[0] // 2, 0)
    db = _fused_specs(n, degp.shape[0] // 2, off)
    out = pl.pallas_call(
        _combine_next_body,
        grid=(1,),
        in_specs=[_full((n, w_in))] + ab + db
        + [_full((1, w_in)), _full((w_out, w_in)), _full((128, w_in))],
        out_specs=[_full((n, w_out)), _full((n, 128))],
        out_shape=[jax.ShapeDtypeStruct((n, w_out), F32),
                   jax.ShapeDtypeStruct((n, 128), F32)],
    )(xs, agg, agg, degp, degp, b.reshape(1, -1), ws, wn_pad)
    return out


def _combine_nlf_body(xs_ref, agg0_ref, agg1_ref, deg0_ref, deg1_ref,
                      b_ref, resp_ref, resb_ref,
                      f1a_ref, f1b_ref, f1bb_ref, f2_ref, f2b_ref, o_ref):
    n, w_in = xs_ref.shape
    agg = agg0_ref[:, :w_in] + agg1_ref[:, :w_in]
    deg = jnp.maximum(deg0_ref[:, 0:1] + deg1_ref[:, 0:1], 1.0)
    h = _elu(xs_ref[...] + agg / deg + b_ref[...])
    res = _elu(resp_ref[...] + resb_ref[...])
    t = _relu(_dot(h, f1a_ref[...]) + _dot(res, f1b_ref[...]) + f1bb_ref[...])
    att = _sigmoid(_dot(t, f2_ref[...]) + f2b_ref[...])
    o_ref[...] = att * h + (1.0 - att) * res


def _rep_gate(w, b, lanes=64):
    return jnp.tile(w, (lanes, 1)), jnp.tile(b.reshape(1, 1), (1, lanes))


def _combine_nlf(xs, agg, degp, off, b, res_pre, res_b, p, name):
    n, w = xs.shape
    f2, f2b = _rep_gate(p[name + '_fc2_w'], p[name + '_fc2_b'], w)
    ab = _fused_specs(n, agg.shape[0] // 2, 0)
    db = _fused_specs(n, degp.shape[0] // 2, off)
    return pl.pallas_call(
        _combine_nlf_body,
        grid=(1,),
        in_specs=[_full((n, w))] + ab + db
        + [_full((1, w)), _full((n, w)), _full((1, w)),
           _full((w, w)), _full((w, w)), _full((1, w)),
           _full((w, w)), _full((1, w))],
        out_specs=_full((n, w)),
        out_shape=jax.ShapeDtypeStruct((n, w), F32),
    )(xs, agg, agg, degp, degp, b.reshape(1, -1), res_pre,
      res_b.reshape(1, -1),
      p[name + '_fc1_w'][:, :w], p[name + '_fc1_w'][:, w:],
      p[name + '_fc1_b'].reshape(1, -1),
      f2, f2b)


def _nlf_gf_body(x1_ref, x2_ref, f1a_ref, f1b_ref, f1bb_ref, f2_ref, f2b_ref,
                 g1_ref, g1b_ref, g2_ref, g2b_ref, o_ref):
    x1 = x1_ref[...]
    x2 = x2_ref[...]
    t = _relu(_dot(x1, f1a_ref[...]) + _dot(x2, f1b_ref[...]) + f1bb_ref[...])
    att = _sigmoid(_dot(t, f2_ref[...]) + f2b_ref[...])
    y = att * x1 + (1.0 - att) * x2
    gc = jnp.mean(y, axis=0, keepdims=True)
    hg = _relu(_dot(gc, g1_ref[...]) + g1b_ref[...])
    ag = _sigmoid(_dot(hg, g2_ref[...]) + g2b_ref[...])
    o_ref[...] = ag * y + (1.0 - ag) * gc


def _nlf_gf(x1, x2, p, nlf_name, gf_name):
    n, w = x1.shape
    f2, f2b = _rep_gate(p[nlf_name + '_fc2_w'], p[nlf_name + '_fc2_b'], w)
    g2, g2b = _rep_gate(p[gf_name + '_fc2_w'], p[gf_name + '_fc2_b'], w)
    return pl.pallas_call(
        _nlf_gf_body,
        out_shape=jax.ShapeDtypeStruct((n, w), F32),
    )(x1, x2,
      p[nlf_name + '_fc1_w'][:, :w], p[nlf_name + '_fc1_w'][:, w:],
      p[nlf_name + '_fc1_b'].reshape(1, -1),
      f2, f2b,
      p[gf_name + '_fc1_w'], p[gf_name + '_fc1_b'].reshape(1, -1),
      g2, g2b)


def _mlp_body(x_ref, w0_ref, b0_ref, w1_ref, b1_ref, o_ref):
    h = _elu(_dot(x_ref[...], w0_ref[...]) + b0_ref[...])
    o_ref[...] = _sigmoid(_dot(h, w1_ref[...]) + b1_ref[...])


def _mlp(x, w0, b0, w1, b1, bm=2048):
    m, k = x.shape
    h0 = w0.shape[0]
    return pl.pallas_call(
        _mlp_body,
        grid=(m // bm,),
        in_specs=[pl.BlockSpec((bm, k), lambda i: (i, 0)),
                  pl.BlockSpec((h0, k), lambda i: (0, 0)),
                  pl.BlockSpec((1, h0), lambda i: (0, 0)),
                  pl.BlockSpec((128, h0), lambda i: (0, 0)),
                  pl.BlockSpec((1, 128), lambda i: (0, 0))],
        out_specs=pl.BlockSpec((bm, 128), lambda i: (i, 0)),
        out_shape=jax.ShapeDtypeStruct((m, 128), F32),
    )(x, w0, b0.reshape(1, -1), *_rep_gate(w1, b1, 128))[:, :1]


_SC_MESH = plsc.VectorSubcoreMesh(core_axis_name="c", subcore_axis_name="s",
                                  num_cores=NC, num_subcores=NS)


@functools.lru_cache(maxsize=None)
def _segsum_call(n_nodes, width, n_edges):
    NSLOT = 3
    ew = n_edges // NW
    nch = ew // K_CH
    rps = n_nodes // NS
    assert ew * NW == n_edges and nch * K_CH == ew and rps * NS == n_nodes

    scratch = (
        [pltpu.VMEM((K_CH,), jnp.int32) for _ in range(nch)]
        + [pltpu.VMEM((K_CH,), jnp.int32) for _ in range(nch)]
        + [pltpu.VMEM((NSLOT, K_CH, width), F32),
           pltpu.VMEM_SHARED((n_nodes, width), F32)]
        + [pltpu.SemaphoreType.DMA]
        + [pltpu.SemaphoreType.DMA for _ in range(NSLOT)]
        + [pltpu.SemaphoreType.DMA for _ in range(NSLOT)])

    def body(xw_hbm, src_hbm, dst_hbm, z_hbm, agg_hbm, *scr):
        sidx = scr[:nch]
        didx = scr[nch:2 * nch]
        rows, sh_agg, isem = scr[2 * nch:2 * nch + 3]
        gsem = scr[2 * nch + 3:2 * nch + 3 + NSLOT]
        ssem = scr[2 * nch + 3 + NSLOT:2 * nch + 3 + 2 * NSLOT]
        c = lax.axis_index("c")
        s = lax.axis_index("s")
        wid = c * NS + s
        r0 = s * rps
        pltpu.sync_copy(z_hbm, sh_agg.at[pl.ds(r0, rps)])
        plsc.subcore_barrier()

        base = wid * ew

        ih = [pltpu.async_copy(src_hbm.at[pl.ds(base + i * K_CH, K_CH)],
                               sidx[i], isem) for i in range(nch)]
        ih += [pltpu.async_copy(dst_hbm.at[pl.ds(base + i * K_CH, K_CH)],
                                didx[i], isem) for i in range(nch)]
        for h in ih:
            h.wait()

        gh = [None] * NSLOT
        for b in range(min(NSLOT, nch)):
            gh[b] = pltpu.async_copy(xw_hbm.at[sidx[b]], rows.at[b], gsem[b])
        tail_s = [None] * NSLOT
        for i in range(nch):
            b = i % NSLOT
            gh[b].wait()
            sh = pltpu.async_copy(rows.at[b], sh_agg.at[didx[i]],
                                  ssem[b], add=True)
            j = i + NSLOT
            if j < nch:
                sh.wait()
                gh[b] = pltpu.async_copy(xw_hbm.at[sidx[j]], rows.at[b],
                                         gsem[b])
            else:
                tail_s[b] = sh
        for h in tail_s:
            if h is not None:
                h.wait()
        plsc.subcore_barrier()
        pltpu.sync_copy(sh_agg.at[pl.ds(r0, rps)],
                        agg_hbm.at[pl.ds(c * n_nodes + r0, rps)])

    return pl.kernel(
        body, out_type=jax.ShapeDtypeStruct((2 * n_nodes, width), F32),
        mesh=_SC_MESH, scratch_types=scratch)


def _segsum(xw, src, dst, n_nodes):
    n_edges = src.shape[0]
    width = xw.shape[1]
    z = jnp.zeros((n_nodes // NS, width), F32)
    return _segsum_call(n_nodes, width, n_edges)(xw, src, dst, z)


@functools.lru_cache(maxsize=None)
def _deg_call(n_nodes, n_edges):
    NSLOT = 4
    ew = n_edges // NW
    nch = ew // K_CH
    rps = n_nodes // NS
    assert ew * NW == n_edges and nch * K_CH == ew and rps * NS == n_nodes

    def body(dst_hbm, z_hbm, ones_hbm, deg_hbm, *scr):
        didx = scr[:nch]
        ones_v, sh_deg, isem = scr[nch:nch + 3]
        dsem = scr[nch + 3:]
        c = lax.axis_index("c")
        s = lax.axis_index("s")
        wid = c * NS + s
        r0 = s * rps
        pltpu.sync_copy(z_hbm, sh_deg.at[pl.ds(r0, rps)])
        pltpu.sync_copy(ones_hbm, ones_v)
        plsc.subcore_barrier()

        base = wid * ew
        ih = [pltpu.async_copy(dst_hbm.at[pl.ds(base + i * K_CH, K_CH)],
                               didx[i], isem) for i in range(nch)]
        for h in ih:
            h.wait()
        dh = [None] * NSLOT
        for i in range(nch):
            b = i % NSLOT
            if dh[b] is not None:
                dh[b].wait()
            dh[b] = pltpu.async_copy(ones_v, sh_deg.at[didx[i]],
                                     dsem[b], add=True)
        for h in dh:
            if h is not None:
                h.wait()
        plsc.subcore_barrier()
        pltpu.sync_copy(sh_deg.at[pl.ds(r0, rps)],
                        deg_hbm.at[pl.ds(c * n_nodes + r0, rps)])

    return pl.kernel(
        body, out_type=jax.ShapeDtypeStruct((2 * n_nodes, 128), F32),
        mesh=_SC_MESH,
        scratch_types=(
            [pltpu.VMEM((K_CH,), jnp.int32) for _ in range(nch)]
            + [pltpu.VMEM((K_CH, 128), F32),
               pltpu.VMEM_SHARED((n_nodes, 128), F32),
               pltpu.SemaphoreType.DMA]
            + [pltpu.SemaphoreType.DMA for _ in range(NSLOT)]))


def _deg(dst, n_nodes):
    z = jnp.zeros((n_nodes // NS, 128), F32)
    ones = jnp.ones((K_CH, 128), F32)
    return _deg_call(n_nodes, dst.shape[0])(dst, z, ones)


@functools.lru_cache(maxsize=None)
def _gather_call(n_rows, n_tab, width):
    NSLOT = 4
    per_w = n_rows // NW
    nch = per_w // K_CH
    assert per_w * NW == n_rows and nch * K_CH == per_w

    def body(tab_hbm, idx_hbm, out_hbm, *scr):
        idxs = scr[:nch]
        rows, isem = scr[nch:nch + 2]
        gsem = scr[nch + 2:nch + 2 + NSLOT]
        wsem = scr[nch + 2 + NSLOT:]
        wid = lax.axis_index("c") * NS + lax.axis_index("s")
        base = wid * per_w

        ih = [pltpu.async_copy(idx_hbm.at[pl.ds(base + i * K_CH, K_CH)],
                               idxs[i], isem) for i in range(nch)]
        for h in ih:
            h.wait()

        gh = [None] * NSLOT
        for b in range(min(NSLOT, nch)):
            gh[b] = pltpu.async_copy(tab_hbm.at[idxs[b]], rows.at[b], gsem[b])
        tail = [None] * NSLOT
        for i in range(nch):
            b = i % NSLOT
            gh[b].wait()
            wh = pltpu.async_copy(rows.at[b],
                                  out_hbm.at[pl.ds(base + i * K_CH, K_CH)],
                                  wsem[b])
            j = i + NSLOT
            if j < nch:
                wh.wait()
                gh[b] = pltpu.async_copy(tab_hbm.at[idxs[j]], rows.at[b],
                                         gsem[b])
            else:
                tail[b] = wh
        for h in tail:
            if h is not None:
                h.wait()

    return pl.kernel(
        body,
        out_type=jax.ShapeDtypeStruct((n_rows, width), F32),
        mesh=_SC_MESH,
        scratch_types=(
            [pltpu.VMEM((K_CH,), jnp.int32) for _ in range(nch)]
            + [pltpu.VMEM((NSLOT, K_CH, width), F32),
               pltpu.SemaphoreType.DMA]
            + [pltpu.SemaphoreType.DMA for _ in range(NSLOT)]
            + [pltpu.SemaphoreType.DMA for _ in range(NSLOT)]))


def _gather_rows(table, idx):
    return _gather_call(idx.shape[0], table.shape[0], table.shape[1])(
        table, idx)


def _pad_rows(w, to=128):
    n, k = w.shape
    if n == to:
        return w
    return jnp.concatenate([w, jnp.zeros((to - n, k), F32)], axis=0)


def kernel(miRNA, disease, img, params, mm_edge_index, dd_edge_index,
           md_edge_index, samples, triplet_samples):
    p = params
    nm = miRNA.shape[0]
    nd = disease.shape[0]

    wm = jnp.concatenate([p['lin_m_w'], p['res1_w'],
                          p['mm1_ws'], p['mm1_wn']], axis=0)
    wd = jnp.concatenate([p['lin_d_w'], p['res2_w'],
                          p['dd1_ws'], p['dd1_wn']], axis=0)
    am = _matmul(miRNA, wm)
    ad = _matmul(disease, wd)
    lm, resm_pre, xs1_m, xw1_m = (am[:, :512], am[:, 512:576],
                                  am[:, 576:704], am[:, 704:832])
    ld, resd_pre, xs1_d, xw1_d = (ad[:, :512], ad[:, 512:576],
                                  ad[:, 576:704], ad[:, 704:832])

    md_feat = jnp.concatenate([lm, ld], axis=0)
    wmd = jnp.concatenate([p['res3_w'], p['md1_ws'], p['md1_wn']], axis=0)
    amd = _matmul(md_feat, wmd)
    resmd_pre, xs1_g, xw1_g = amd[:, :64], amd[:, 64:192], amd[:, 192:320]

    ntot = 2 * (nm + nd)
    o_mm, o_dd, o_md = 0, nm, nm + nd
    cdst = jnp.concatenate([mm_edge_index[1], dd_edge_index[1] + o_dd,
                            md_edge_index[1] + o_md])
    degp = _deg(cdst, ntot)

    def chain(xs1, xw1, src, dst, n, off, pre, res_pre, res_b, nlf_name):
        agg1 = _segsum(xw1, src, dst, n)
        ys2, yw2 = _combine_next(xs1, agg1, degp, off, p[pre + '1_b'],
                                 p[pre + '2_ws'], _pad_rows(p[pre + '2_wn']))
        agg2 = _segsum(yw2, src, dst, n)
        ys3, yw3 = _combine_next(ys2, agg2, degp, off, p[pre + '2_b'],
                                 p[pre + '3_ws'], _pad_rows(p[pre + '3_wn']))
        agg3 = _segsum(yw3, src, dst, n)
        return _combine_nlf(ys3, agg3, degp, off, p[pre + '3_b'],
                            res_pre, res_b, p, nlf_name)

    emb_mm_sim = chain(xs1_m, xw1_m, mm_edge_index[0], mm_edge_index[1],
                       nm, o_mm, 'mm', resm_pre, p['res1_b'], 'nlf_m')
    emb_dd_sim = chain(xs1_d, xw1_d, dd_edge_index[0], dd_edge_index[1],
                       nd, o_dd, 'dd', resd_pre, p['res2_b'], 'nlf_d')
    emb_ass = chain(xs1_g, xw1_g, md_edge_index[0], md_edge_index[1],
                    nm + nd, o_md, 'md', resmd_pre, p['res3_b'], 'nlf_md')

    emb_mm = _nlf_gf(emb_mm_sim, emb_ass[:nm], p, 'nlf_m', 'gf_m')
    emb_dd = _nlf_gf(emb_dd_sim, emb_ass[nm:], p, 'nlf_d', 'gf_d')

    emb_cat = jnp.concatenate([
        jnp.concatenate([emb_mm, emb_dd], axis=0),
        jnp.zeros((nm + nd, 64), F32)], axis=1)
    off = jnp.int32(nm)
    idx_cat = jnp.concatenate([
        samples[:, 0], samples[:, 1] + off,
        triplet_samples[:, 0], triplet_samples[:, 1] + off,
        triplet_samples[:, 2] + off])
    g = _gather_rows(emb_cat, idx_cat)[:, :64]
    ns = samples.shape[0]
    gm, gd = g[:ns], g[ns:2 * ns]
    anchor = g[2 * ns:3 * ns]
    pos = g[3 * ns:4 * ns]
    neg = g[4 * ns:5 * ns]

    emb = jnp.concatenate([gm, gd, img[:ns]], axis=1)
    out = _mlp(emb, p['mlp0_w'], p['mlp0_b'], p['mlp1_w'], p['mlp1_b'])
    return (out, anchor, pos, neg)
